# Initial kernel scaffold; baseline (speedup 1.0000x reference)
import jax
import jax.numpy as jnp
from jax.experimental import pallas as pl


def kernel(x, inc_v_pos, inc_e_pos, inc_v_neg, inc_e_neg, W_pos, b_pos, W_neg, b_neg):
    raise NotImplementedError("write your pallas kernel here")


# speedup vs baseline: 1.8630x; 1.8630x over previous
import functools

import jax
import jax.numpy as jnp
from jax import lax
from jax.experimental import pallas as pl
from jax.experimental.pallas import tpu as pltpu
from jax.experimental.pallas import tpu_sc as plsc

NV = 10000
NE = 5000
NNZ = 160000
D = 256
WS = 128
NST = D // WS
DW = 16
NC = 2
NS = 16
PAIRS = NNZ // NS
CH = 128
NCH = PAIRS // CH
REM = PAIRS - NCH * CH
ACC = 5120
ACCV = 10240
ZR = 64

_MESH = plsc.VectorSubcoreMesh(
    core_axis_name="c", subcore_axis_name="s", num_cores=NC, num_subcores=NS)

F32 = jnp.float32
I32 = jnp.int32


def _fill_f32(ref, nrows, ncols, value):
  def row(i, _):
    def col(j, _):
      ref[i, pl.ds(j * 16, 16)] = jnp.full((16,), value, F32)
      return 0
    return lax.fori_loop(0, ncols // 16, col, 0)
  lax.fori_loop(0, nrows, row, 0)


def _fill_i32(ref, n, value):
  def g(i, _):
    ref[pl.ds(i * 16, 16)] = jnp.full((16,), value, I32)
    return 0
  lax.fori_loop(0, n // 16, g, 0)


def _addc_i32(src, dst, cst):
  def g(i, _):
    dst[pl.ds(i * 16, 16)] = src[pl.ds(i * 16, 16)] + cst
    return 0
  lax.fori_loop(0, CH // 16, g, 0)


def _remap_i32(src, dst, h):
  def g(i, _):
    v = src[pl.ds(i * 16, 16)]
    w = v - h * NE
    m = jnp.logical_and(w >= 0, w < NE)
    dst[pl.ds(i * 16, 16)] = jnp.where(m, w, jnp.full((16,), NE, I32))
    return 0
  lax.fori_loop(0, CH // 16, g, 0)


def _sc_degree_pass(idxp, idxn, n, nhalf):
  @functools.partial(
      pl.kernel,
      out_type=(jax.ShapeDtypeStruct((n, WS), F32),
                jax.ShapeDtypeStruct((n, WS), F32)),
      mesh=_MESH,
      scratch_types=[
          pltpu.VMEM((CH,), I32),
          pltpu.VMEM((CH,), I32),
          pltpu.VMEM((CH, WS), F32),
          pltpu.VMEM((ACC // NS, WS), F32),
          pltpu.VMEM_SHARED((ACC, WS), F32),
      ],
  )
  def k(ip_h, in_h, dp_h, dn_h, ib, jb, ones, zbuf, acc):
    c = lax.axis_index("c")
    s = lax.axis_index("s")
    base = s * PAIRS
    _fill_f32(ones, CH, WS, 1.0)
    _fill_f32(zbuf, ACC // NS, WS, 0.0)

    def run(idx_h, out_h):
      for h in range(nhalf):
        pltpu.sync_copy(zbuf, acc.at[pl.ds(s * (ACC // NS), ACC // NS)])
        plsc.subcore_barrier()
        def chunk(i, _):
          pltpu.sync_copy(idx_h.at[pl.ds(base + i * CH, CH)], ib)
          _remap_i32(ib, jb, h)
          pltpu.sync_copy(ones, acc.at[jb], add=True)
          return 0
        lax.fori_loop(0, NCH, chunk, 0)
        _fill_i32(ib, CH, NV)
        pltpu.sync_copy(idx_h.at[pl.ds(base + NCH * CH, REM)],
                        ib.at[pl.ds(0, REM)])
        _remap_i32(ib, jb, h)
        pltpu.sync_copy(ones, acc.at[jb], add=True)
        plsc.subcore_barrier()
        obase = h * NE
        @pl.when(s < NS - 1)
        def _():
          pltpu.sync_copy(acc.at[pl.ds(s * 320, 320)],
                          out_h.at[pl.ds(obase + s * 320, 320)])
        @pl.when(s == NS - 1)
        def _():
          pltpu.sync_copy(acc.at[pl.ds((NS - 1) * 320, 200)],
                          out_h.at[pl.ds(obase + (NS - 1) * 320, 200)])

    @pl.when(c == 0)
    def _():
      run(ip_h, dp_h)
    @pl.when(c == 1)
    def _():
      run(in_h, dn_h)

  return k(idxp, idxn)


def _sc_degrees_unused(ivp, iep, ivn, ien):
  @functools.partial(
      pl.kernel,
      out_type=(jax.ShapeDtypeStruct((NV, WS), F32),
                jax.ShapeDtypeStruct((NE, WS), F32),
                jax.ShapeDtypeStruct((NV, WS), F32),
                jax.ShapeDtypeStruct((NE, WS), F32)),
      mesh=_MESH,
      scratch_types=[
          pltpu.VMEM((CH,), I32),
          pltpu.VMEM((CH, WS), F32),
          pltpu.VMEM((ACCV // NS, WS), F32),
          pltpu.VMEM_SHARED((ACCV, WS), F32),
          pltpu.VMEM_SHARED((NE + DW, WS), F32),
      ],
  )
  def k(ivp_h, iep_h, ivn_h, ien_h, dvp_h, dep_h, dvn_h, den_h,
        ib, ones, zbuf, accv, acce):
    c = lax.axis_index("c")
    s = lax.axis_index("s")
    base = s * PAIRS
    _fill_f32(ones, CH, WS, 1.0)
    _fill_f32(zbuf, ACCV // NS, WS, 0.0)
    pltpu.sync_copy(zbuf, accv.at[pl.ds(s * (ACCV // NS), ACCV // NS)])
    @pl.when(s == 0)
    def _():
      pltpu.sync_copy(zbuf.at[pl.ds(0, DW)], acce.at[pl.ds(NE, DW)])
    @pl.when(s < NS - 1)
    def _():
      pltpu.sync_copy(zbuf.at[pl.ds(0, 320)], acce.at[pl.ds(s * 320, 320)])
    @pl.when(s == NS - 1)
    def _():
      pltpu.sync_copy(zbuf.at[pl.ds(0, 200)], acce.at[pl.ds(4800, 200)])
    plsc.subcore_barrier()

    def count(idx_h, acc, trash):
      def chunk(i, _):
        pltpu.sync_copy(idx_h.at[pl.ds(base + i * CH, CH)], ib)
        pltpu.sync_copy(ones, acc.at[ib], add=True)
        return 0
      lax.fori_loop(0, NCH, chunk, 0)
      _fill_i32(ib, CH, trash)
      pltpu.sync_copy(idx_h.at[pl.ds(base + NCH * CH, REM)],
                      ib.at[pl.ds(0, REM)])
      pltpu.sync_copy(ones, acc.at[ib], add=True)

    @pl.when(c == 0)
    def _():
      count(ivp_h, accv, NV)
      count(iep_h, acce, NE)
    @pl.when(c == 1)
    def _():
      count(ivn_h, accv, NV)
      count(ien_h, acce, NE)
    plsc.subcore_barrier()

    def copy_out(acc, out_h, rows, rows_last):
      @pl.when(s < NS - 1)
      def _():
        pltpu.sync_copy(acc.at[pl.ds(s * rows, rows)],
                        out_h.at[pl.ds(s * rows, rows)])
      @pl.when(s == NS - 1)
      def _():
        pltpu.sync_copy(acc.at[pl.ds((NS - 1) * rows, rows_last)],
                        out_h.at[pl.ds((NS - 1) * rows, rows_last)])

    @pl.when(c == 0)
    def _():
      copy_out(accv, dvp_h, 640, 400)
      copy_out(acce, dep_h, 320, 200)
    @pl.when(c == 1)
    def _():
      copy_out(accv, dvn_h, 640, 400)
      copy_out(acce, den_h, 320, 200)

  return k(ivp, iep, ivn, ien)


def _sc_segsum(tab_p, tab_n, gp, sp, gn, sn, nsrc, ndst):
  nhalf = ndst // NE

  @functools.partial(
      pl.kernel,
      out_type=(jax.ShapeDtypeStruct((NST * ndst, WS), F32),
                jax.ShapeDtypeStruct((NST * ndst, WS), F32)),
      mesh=_MESH,
      scratch_types=[
          pltpu.VMEM((CH,), I32),
          pltpu.VMEM((CH,), I32),
          pltpu.VMEM((CH,), I32),
          pltpu.VMEM((CH,), I32),
          pltpu.VMEM((CH, WS), F32),
          pltpu.VMEM((ZR, WS), F32),
          pltpu.VMEM_SHARED((ACC, WS), F32),
          pltpu.SemaphoreType.DMA,
      ],
  )
  def k(tp_h, tn_h, gp_h, sp_h, gn_h, sn_h, outp_h, outn_h,
        gi, gj, si, sj, rows, zbuf, acc, sem):
    c = lax.axis_index("c")
    s = lax.axis_index("s")
    base = s * PAIRS
    _fill_f32(zbuf, ZR, WS, 0.0)

    def run(tab_h, g_h, s_h, out_h):
      for f in range(NST):
        for h in range(nhalf):
          def z(i, _):
            pltpu.sync_copy(zbuf, acc.at[pl.ds(s * (ACC // NS) + i * ZR, ZR)])
            return 0
          lax.fori_loop(0, (ACC // NS) // ZR, z, 0)
          plsc.subcore_barrier()

          def scatter():
            if nhalf == 1:
              pltpu.sync_copy(rows, acc.at[si], add=True)
            else:
              _remap_i32(si, sj, h)
              pltpu.sync_copy(rows, acc.at[sj], add=True)

          def chunk(i, _):
            off = base + i * CH
            pltpu.sync_copy(g_h.at[pl.ds(off, CH)], gi)
            pltpu.sync_copy(s_h.at[pl.ds(off, CH)], si)
            _addc_i32(gi, gj, f * nsrc)
            pltpu.async_copy(tab_h.at[gj], rows, sem).wait()
            scatter()
            return 0
          lax.fori_loop(0, NCH, chunk, 0)
          _fill_i32(gi, CH, 0)
          _fill_i32(si, CH, ndst)
          off = base + NCH * CH
          pltpu.sync_copy(g_h.at[pl.ds(off, REM)], gi.at[pl.ds(0, REM)])
          pltpu.sync_copy(s_h.at[pl.ds(off, REM)], si.at[pl.ds(0, REM)])
          _addc_i32(gi, gj, f * nsrc)
          pltpu.async_copy(tab_h.at[gj], rows, sem).wait()
          scatter()
          plsc.subcore_barrier()
          obase = f * ndst + h * NE
          @pl.when(s < NS - 1)
          def _():
            pltpu.sync_copy(acc.at[pl.ds(s * 320, 320)],
                            out_h.at[pl.ds(obase + s * 320, 320)])
          @pl.when(s == NS - 1)
          def _():
            pltpu.sync_copy(acc.at[pl.ds((NS - 1) * 320, 200)],
                            out_h.at[pl.ds(obase + (NS - 1) * 320, 200)])

    @pl.when(c == 0)
    def _():
      run(tp_h, gp_h, sp_h, outp_h)
    @pl.when(c == 1)
    def _():
      run(tn_h, gn_h, sn_h, outn_h)

  return k(tab_p, tab_n, gp, sp, gn, sn)


def _tc_linear(x, Wp, bp, dvp, Wn, bn, dvn):
  bm = 400
  nb = NV // bm
  def body(x_r, wp_r, bp_r, dp_r, wn_r, bn_r, dn_r, op_r, on_r):
    xb = x_r[...]
    dp = dp_r[:, 0:1]
    dn = dn_r[:, 0:1]
    sp = jnp.where(dp > 0, lax.rsqrt(jnp.maximum(dp, 1e-30)), 0.0)
    sn = jnp.where(dn > 0, lax.rsqrt(jnp.maximum(dn, 1e-30)), 0.0)
    zp = lax.dot_general(xb, wp_r[...], (((1,), (1,)), ((), ())),
                         preferred_element_type=F32)
    zn = lax.dot_general(xb, wn_r[...], (((1,), (1,)), ((), ())),
                         preferred_element_type=F32)
    zp = (zp + bp_r[...]) * sp
    zn = (zn + bn_r[...]) * sn
    for f in range(NST):
      op_r[f, :, :] = zp[:, f * WS:(f + 1) * WS]
      on_r[f, :, :] = zn[:, f * WS:(f + 1) * WS]
  return pl.pallas_call(
      body,
      grid=(nb,),
      in_specs=[
          pl.BlockSpec((bm, D), lambda j: (j, 0)),
          pl.BlockSpec((D, D), lambda j: (0, 0)),
          pl.BlockSpec((1, D), lambda j: (0, 0)),
          pl.BlockSpec((bm, WS), lambda j: (j, 0)),
          pl.BlockSpec((D, D), lambda j: (0, 0)),
          pl.BlockSpec((1, D), lambda j: (0, 0)),
          pl.BlockSpec((bm, WS), lambda j: (j, 0)),
      ],
      out_specs=[pl.BlockSpec((NST, bm, WS), lambda j: (0, j, 0)),
                 pl.BlockSpec((NST, bm, WS), lambda j: (0, j, 0))],
      out_shape=[jax.ShapeDtypeStruct((NST, NV, WS), F32),
                 jax.ShapeDtypeStruct((NST, NV, WS), F32)],
  )(x, Wp, bp.reshape(1, D), dvp, Wn, bn.reshape(1, D), dvn)


def _tc_scale_edges(Sp_raw, Sn_raw, dep, den):
  bm = 200
  nb = NE // bm
  def body(sp_r, sn_r, dp_r, dn_r, op_r, on_r):
    dp = dp_r[:, 0:1]
    dn = dn_r[:, 0:1]
    ip_ = jnp.where(dp > 0, 1.0 / jnp.maximum(dp, 1e-30), 0.0)
    in_ = jnp.where(dn > 0, 1.0 / jnp.maximum(dn, 1e-30), 0.0)
    op_r[...] = sp_r[...] * ip_
    on_r[...] = sn_r[...] * in_
  return pl.pallas_call(
      body,
      grid=(NST, nb),
      in_specs=[
          pl.BlockSpec((bm, WS), lambda f, j: (f * nb + j, 0)),
          pl.BlockSpec((bm, WS), lambda f, j: (f * nb + j, 0)),
          pl.BlockSpec((bm, WS), lambda f, j: (j, 0)),
          pl.BlockSpec((bm, WS), lambda f, j: (j, 0)),
      ],
      out_specs=[pl.BlockSpec((bm, WS), lambda f, j: (f * nb + j, 0))] * 2,
      out_shape=[jax.ShapeDtypeStruct((NST * NE, WS), F32)] * 2,
  )(Sp_raw, Sn_raw, dep, den)


def _tc_nonlinear(Ep_raw, En_raw, dvp, dvn, x):
  bm = 400
  nb = NV // bm
  def body(ep_r, en_r, dp_r, dn_r, x_r, oep_r, oen_r, node_r):
    dp = dp_r[:, 0:1]
    dn = dn_r[:, 0:1]
    sp = jnp.where(dp > 0, lax.rsqrt(jnp.maximum(dp, 1e-30)), 0.0)
    sn = jnp.where(dn > 0, lax.rsqrt(jnp.maximum(dn, 1e-30)), 0.0)
    for f in range(NST):
      ep = ep_r[f, :, :] * sp
      en = en_r[f, :, :] * sn
      ep = jnp.where(ep >= 0, ep, 0.01 * ep)
      en = jnp.where(en >= 0, en, 0.01 * en)
      oep_r[f, :, :] = ep
      oen_r[f, :, :] = en
      node_r[:, f * WS:(f + 1) * WS] = (
          x_r[:, f * WS:(f + 1) * WS] * 0.5 + (ep + en) * 0.25)
  return pl.pallas_call(
      body,
      grid=(nb,),
      in_specs=[
          pl.BlockSpec((NST, bm, WS), lambda j: (0, j, 0)),
          pl.BlockSpec((NST, bm, WS), lambda j: (0, j, 0)),
          pl.BlockSpec((bm, WS), lambda j: (j, 0)),
          pl.BlockSpec((bm, WS), lambda j: (j, 0)),
          pl.BlockSpec((bm, D), lambda j: (j, 0)),
      ],
      out_specs=[pl.BlockSpec((NST, bm, WS), lambda j: (0, j, 0)),
                 pl.BlockSpec((NST, bm, WS), lambda j: (0, j, 0)),
                 pl.BlockSpec((bm, D), lambda j: (j, 0))],
      out_shape=[jax.ShapeDtypeStruct((NST, NV, WS), F32),
                 jax.ShapeDtypeStruct((NST, NV, WS), F32),
                 jax.ShapeDtypeStruct((NV, D), F32)],
  )(Ep_raw, En_raw, dvp, dvn, x)


def _tc_review(Op_raw, On_raw, dep, den):
  bm = 200
  nb = NE // bm
  def body(op_r, on_r, dp_r, dn_r, out_r):
    dp = dp_r[:, 0:1]
    dn = dn_r[:, 0:1]
    ip_ = jnp.where(dp > 0, 1.0 / jnp.maximum(dp, 1e-30), 0.0)
    in_ = jnp.where(dn > 0, 1.0 / jnp.maximum(dn, 1e-30), 0.0)
    for f in range(NST):
      out_r[:, f * WS:(f + 1) * WS] = (
          op_r[f, :, :] * ip_ + on_r[f, :, :] * in_) * 0.5
  return pl.pallas_call(
      body,
      grid=(nb,),
      in_specs=[
          pl.BlockSpec((NST, bm, WS), lambda j: (0, j, 0)),
          pl.BlockSpec((NST, bm, WS), lambda j: (0, j, 0)),
          pl.BlockSpec((bm, WS), lambda j: (j, 0)),
          pl.BlockSpec((bm, WS), lambda j: (j, 0)),
      ],
      out_specs=pl.BlockSpec((bm, D), lambda j: (j, 0)),
      out_shape=jax.ShapeDtypeStruct((NE, D), F32),
  )(Op_raw, On_raw, dep, den)


def kernel(x, inc_v_pos, inc_e_pos, inc_v_neg, inc_e_neg,
           W_pos, b_pos, W_neg, b_neg):
  dvp, dvn = _sc_degree_pass(inc_v_pos, inc_v_neg, NV, 2)
  dep, den = _sc_degree_pass(inc_e_pos, inc_e_neg, NE, 1)
  Tp, Tn = _tc_linear(x, W_pos, b_pos, dvp, W_neg, b_neg, dvn)
  Sp_raw, Sn_raw = _sc_segsum(
      Tp.reshape(NST * NV, WS), Tn.reshape(NST * NV, WS),
      inc_v_pos, inc_e_pos, inc_v_neg, inc_e_neg, NV, NE)
  Sp, Sn = _tc_scale_edges(Sp_raw, Sn_raw, dep, den)
  Ep_raw, En_raw = _sc_segsum(
      Sp, Sn, inc_e_pos, inc_v_pos, inc_e_neg, inc_v_neg, NE, NV)
  ep, en, node = _tc_nonlinear(Ep_raw.reshape(NST, NV, WS),
                               En_raw.reshape(NST, NV, WS), dvp, dvn, x)
  Op_raw, On_raw = _sc_segsum(
      ep.reshape(NST * NV, WS), en.reshape(NST * NV, WS),
      inc_v_pos, inc_e_pos, inc_v_neg, inc_e_neg, NV, NE)
  review = _tc_review(Op_raw.reshape(NST, NE, WS),
                      On_raw.reshape(NST, NE, WS), dep, den)
  return (node, review)
